# baseline (speedup 1.0000x reference)
import functools

import jax
import jax.numpy as jnp
from jax import lax
from jax.experimental import pallas as pl
from jax.experimental.pallas import tpu as pltpu
from jax.experimental.pallas import tpu_sc as plsc

EMBED = 64
BATCH = 16384
OUT_W = 5 * EMBED

_info = plsc.get_sparse_core_info()
_NW = _info.num_cores * _info.num_subcores
_BPW = BATCH // _NW

_mesh = plsc.VectorSubcoreMesh(core_axis_name="c", subcore_axis_name="s")
_params = pltpu.CompilerParams(use_tc_tiling_on_sc=False)


def _worker_base():
    return (lax.axis_index("s") * _info.num_cores + lax.axis_index("c")) * _BPW


@functools.partial(
    pl.kernel,
    mesh=_mesh,
    out_type=(),
    scratch_types=[
        pltpu.VMEM((4, _BPW), jnp.int32),
        pltpu.VMEM((2, _BPW, EMBED), jnp.float32),
        pltpu.SemaphoreType.DMA,
    ],
    compiler_params=_params,
    name="small_fields",
)
def _small_fields(ep, pop, yr, st, et, pt, yt, stt, out, idx_v, rows_v, gsem):
    base = _worker_base()
    idx_hbm = [ep, pop, yr, st]
    tables = [et, pt, yt, stt]
    for t in range(4):
        pltpu.sync_copy(idx_hbm[t].at[pl.ds(base, _BPW)], idx_v.at[t])

    def start_gather(t, buf):
        return pltpu.async_copy(tables[t].at[idx_v.at[t]], rows_v.at[buf], gsem)

    cp = start_gather(0, 0)
    for t in range(4):
        cp.wait()
        if t + 1 < 4:
            nxt = start_gather(t + 1, (t + 1) % 2)
        pltpu.sync_copy(
            rows_v.at[t % 2],
            out.at[pl.ds(base, _BPW), pl.ds((t + 1) * EMBED, EMBED)],
        )
        if t + 1 < 4:
            cp = nxt


@functools.partial(
    pl.kernel,
    mesh=_mesh,
    out_type=(),
    scratch_types=[
        pltpu.VMEM((_BPW,), jnp.int32),
        pltpu.VMEM((_BPW, EMBED), jnp.float32),
        pltpu.SemaphoreType.DMA,
    ],
    compiler_params=_params,
    name="user_field",
)
def _user_field(uid, ut, out, idx_v, rows_v, gsem):
    base = _worker_base()
    pltpu.sync_copy(uid.at[pl.ds(base, _BPW)], idx_v)
    pltpu.async_copy(ut.at[idx_v], rows_v, gsem).wait()
    pltpu.sync_copy(rows_v, out.at[pl.ds(base, _BPW), pl.ds(0, EMBED)])


def kernel(user_id, episodes, popularity, year, studio,
           user_table, episodes_table, popularity_table, year_table, studio_table):
    o_ref = jax.new_ref(jnp.zeros((BATCH, OUT_W), jnp.float32))
    _small_fields(episodes, popularity, year, studio,
                  episodes_table, popularity_table, year_table, studio_table,
                  o_ref)
    _user_field(user_id, user_table, o_ref)
    return o_ref[...]


# speedup vs baseline: 1.0468x; 1.0468x over previous
import functools

import jax
import jax.numpy as jnp
from jax import lax
from jax.experimental import pallas as pl
from jax.experimental.pallas import tpu as pltpu
from jax.experimental.pallas import tpu_sc as plsc

EMBED = 64
NF = 5
BATCH = 16384
OUT_W = NF * EMBED

_info = plsc.get_sparse_core_info()
_NW = _info.num_cores * _info.num_subcores
_BPW = BATCH // _NW
_CH = 64
_NCH = _BPW // _CH
_G = _CH // 16


def kernel(user_id, episodes, popularity, year, studio,
           user_table, episodes_table, popularity_table, year_table, studio_table):

    @functools.partial(
        pl.kernel,
        mesh=plsc.VectorSubcoreMesh(core_axis_name="c", subcore_axis_name="s"),
        out_type=jax.ShapeDtypeStruct((BATCH, OUT_W), jnp.float32),
        scratch_types=[
            [pltpu.VMEM((_BPW,), jnp.int32) for _ in range(NF)],
            [pltpu.VMEM((_CH, EMBED), jnp.float32) for _ in range(NF)],
            pltpu.VMEM((_CH, OUT_W), jnp.float32),
            pltpu.SemaphoreType.DMA,
        ],
    )
    def run(uid, ep, pop, yr, st, ut, et, pt, yt, stt, out,
            idx_v, rows_v, stage_v, sem):
        wid = lax.axis_index("s") * _info.num_cores + lax.axis_index("c")
        base = wid * _BPW
        idx_hbm = [uid, ep, pop, yr, st]
        tables = [ut, et, pt, yt, stt]

        for t in range(NF):
            pltpu.sync_copy(idx_hbm[t].at[pl.ds(base, _BPW)], idx_v[t])

        def chunk(c, _):
            for t in range(NF):
                def issue(g, _, _t=t):
                    v = idx_v[_t][pl.ds(c * _CH + g * 16, 16)]
                    for lane in range(16):
                        pltpu.async_copy(
                            tables[_t].at[pl.ds(v[lane], 1), :],
                            rows_v[_t].at[pl.ds(g * 16 + lane, 1), :],
                            sem)
                    return 0
                lax.fori_loop(0, _G, issue, 0)

            def drain(g, _):
                for lane in range(16):
                    pltpu.make_async_copy(
                        tables[0].at[pl.ds(0, 1), :],
                        rows_v[0].at[pl.ds(0, 1), :],
                        sem).wait()
                return 0
            lax.fori_loop(0, NF * _G, drain, 0)

            def asm(i, _):
                for t in range(NF):
                    for g in range(EMBED // 16):
                        stage_v[i, pl.ds(t * EMBED + g * 16, 16)] = (
                            rows_v[t][i, pl.ds(g * 16, 16)])
                return 0
            lax.fori_loop(0, _CH, asm, 0)

            pltpu.sync_copy(stage_v, out.at[pl.ds(base + c * _CH, _CH), :])
            return 0

        lax.fori_loop(0, _NCH, chunk, 0)

    return run(user_id, episodes, popularity, year, studio,
               user_table, episodes_table, popularity_table, year_table,
               studio_table)
